# baseline (speedup 1.0000x reference)
import jax
import jax.numpy as jnp
from jax import lax
from jax.experimental import pallas as pl
from jax.experimental.pallas import tpu as pltpu
from jax.experimental.pallas import tpu_sc as plsc

_CLS = 100000
_DIM = 32
_BATCH = 16384
_NC = 2
_NS = 16
_NW = _NC * _NS
_CHUNK = 128
_NHC = 8
_HIST = _NHC * _CHUNK
_CNT_PAD = 100096
_ZCHUNK = _CNT_PAD // _NS // 2
_Q = 4096


def _body(ys_ref, xsT_ref, ct_ref, out_ref,
          idx_v, crow_v, xrow_v, ysq_v, wq_v, cnt_v, w1k_v, ones_v, z_v,
          acc_v, cnt_sh, w_sh, sem, sem_i, sem_h, sem_x):
    c = lax.axis_index("c")
    s = lax.axis_index("s")
    wid = s * _NC + c
    d = c * _NS + s
    lanes = lax.iota(jnp.int32, 16)
    zero16 = jnp.zeros((16,), jnp.float32)

    ccopy = pltpu.async_copy(ct_ref.at[d], crow_v, sem_x)
    icopy = pltpu.async_copy(ys_ref.at[pl.ds(s * _NHC, _NHC)], idx_v, sem_i)
    q0_copies = [
        pltpu.async_copy(xsT_ref.at[d, pl.ds(0, _Q)], xrow_v, sem),
        pltpu.async_copy(ys_ref.at[pl.ds(0, _Q // _CHUNK)], ysq_v, sem),
    ]

    for k in range(_CHUNK // 16):
        ones_v[pl.ds(k * 16, 16)] = zero16 + 1.0

    def zstore(i, carry):
        z_v[pl.ds(i * 16, 16)] = zero16
        return carry

    lax.fori_loop(0, _ZCHUNK // 16, zstore, 0)
    zcopy = pltpu.async_copy(
        z_v, cnt_sh.at[pl.ds(s * 2 * _ZCHUNK, _ZCHUNK)], sem_h)
    pltpu.sync_copy(z_v, cnt_sh.at[pl.ds((s * 2 + 1) * _ZCHUNK, _ZCHUNK)])
    zcopy.wait()
    plsc.subcore_barrier()

    icopy.wait()
    hist_copies = [
        pltpu.async_copy(ones_v, cnt_sh.at[idx_v.at[g]], sem_h, add=True)
        for g in range(_NHC)
    ]
    for h in hist_copies:
        h.wait()
    plsc.subcore_barrier()

    cnt_copies = [
        pltpu.async_copy(cnt_sh.at[idx_v.at[g]],
                         cnt_v.at[pl.ds(g * _CHUNK, _CHUNK)], sem_h)
        for g in range(_NHC)
    ]
    for cc in cnt_copies:
        cc.wait()

    def wstore(i, carry):
        cnt16 = cnt_v[pl.ds(i * 16, 16)]
        w1k_v[pl.ds(i * 16, 16)] = 0.5 / (cnt16 + 1.0)
        return carry

    lax.fori_loop(0, _HIST // 16, wstore, 0)
    pltpu.sync_copy(w1k_v, w_sh.at[pl.ds(s * _HIST, _HIST)])
    plsc.subcore_barrier()

    ccopy.wait()
    acc = zero16
    for q in range(_BATCH // _Q):
        if q == 0:
            stage = q0_copies
        else:
            stage = [
                pltpu.async_copy(
                    xsT_ref.at[d, pl.ds(q * _Q, _Q)], xrow_v, sem_x),
                pltpu.async_copy(
                    ys_ref.at[pl.ds(q * (_Q // _CHUNK), _Q // _CHUNK)],
                    ysq_v, sem_i),
            ]
        stage = stage + [
            pltpu.async_copy(w_sh.at[pl.ds(q * _Q, _Q)], wq_v, sem_h)]
        for sc in stage:
            sc.wait()

        def group(g, a):
            y16 = ysq_v[g >> 3, pl.ds((g & 7) * 16, 16)]
            cv = plsc.load_gather(crow_v, [y16])
            xv = xrow_v[pl.ds(g * 16, 16)]
            w16 = wq_v[pl.ds(g * 16, 16)]
            t = xv - cv
            return a + w16 * t * t

        acc = lax.fori_loop(0, _Q // 16, group, acc)
    acc_v[...] = acc
    pltpu.sync_copy(acc_v, out_ref.at[pl.ds(wid * 16, 16)])


def kernel(xs, ys, center):
    ys2d = ys.astype(jnp.int32).reshape(_BATCH // _CHUNK, _CHUNK)
    xsT = xs.T
    centerT = center.T
    mesh = plsc.VectorSubcoreMesh(
        core_axis_name="c", subcore_axis_name="s", num_cores=_NC)
    out = pl.kernel(
        _body,
        out_type=jax.ShapeDtypeStruct((_NW * 16,), jnp.float32),
        mesh=mesh,
        compiler_params=pltpu.CompilerParams(
            needs_layout_passes=False, use_tc_tiling_on_sc=True),
        scratch_types=[
            pltpu.VMEM((_NHC, _CHUNK), jnp.int32),
            pltpu.VMEM((_CLS,), jnp.float32),
            pltpu.VMEM((_Q,), jnp.float32),
            pltpu.VMEM((_Q // _CHUNK, _CHUNK), jnp.int32),
            pltpu.VMEM((_Q,), jnp.float32),
            pltpu.VMEM((_HIST,), jnp.float32),
            pltpu.VMEM((_HIST,), jnp.float32),
            pltpu.VMEM((_CHUNK,), jnp.float32),
            pltpu.VMEM((_ZCHUNK,), jnp.float32),
            pltpu.VMEM((16,), jnp.float32),
            pltpu.VMEM_SHARED((_CNT_PAD,), jnp.float32),
            pltpu.VMEM_SHARED((_BATCH,), jnp.float32),
            pltpu.SemaphoreType.DMA,
            pltpu.SemaphoreType.DMA,
            pltpu.SemaphoreType.DMA,
            pltpu.SemaphoreType.DMA,
        ],
    )(ys2d, xsT, centerT)
    return jnp.sum(out)
